# baseline (speedup 1.0000x reference)
import functools

import jax
import jax.numpy as jnp
from jax import lax
from jax.experimental import pallas as pl
from jax.experimental.pallas import tpu as pltpu
from jax.experimental.pallas import tpu_sc as plsc

_N_FIELDS = 26
_VOCAB = 100000
_EMB_DIM = 32
_BATCH = 16384
_L = 16

_G = 2


@functools.cache
def _build_sc_kernel():
    info = plsc.get_sparse_core_info()
    nc, ns = info.num_cores, info.num_subcores
    nw = nc * ns
    bpw = _BATCH // nw

    mesh = plsc.VectorSubcoreMesh(core_axis_name="c", subcore_axis_name="s")

    @functools.partial(
        pl.kernel,
        mesh=mesh,
        out_type=jax.ShapeDtypeStruct((_BATCH, _G * _EMB_DIM), jnp.float32),
        scratch_types=[
            pltpu.VMEM((_G, bpw), jnp.int32),
            pltpu.VMEM((bpw,), jnp.int32),
            pltpu.VMEM((_G, bpw, _EMB_DIM), jnp.float32),
            pltpu.SemaphoreType.DMA,
            pltpu.SemaphoreType.DMA,
        ],
        compiler_params=pltpu.CompilerParams(use_tc_tiling_on_sc=False),
    )
    def sc_embed(idx_hbm, tab_hbm, out_hbm, idx_v, rid_v, rows_v, gsem, wsem):
        wid = lax.axis_index("s") * nc + lax.axis_index("c")
        b0 = wid * bpw

        pltpu.sync_copy(idx_hbm.at[:, pl.ds(b0, bpw)], idx_v)

        gd = [None] * _G
        wd = [None] * _G
        for i in range(_G):

            def prep(k, c, i=i):
                sl = pl.ds(k * _L, _L)
                rid_v[sl] = idx_v[i, sl] + i * _VOCAB
                return c

            lax.fori_loop(0, bpw // _L, prep, 0, unroll=4)
            gd[i] = pltpu.async_copy(tab_hbm.at[rid_v], rows_v.at[i], gsem)
            gd[i].wait()
            wd[i] = pltpu.async_copy(
                rows_v.at[i],
                out_hbm.at[pl.ds(b0, bpw), pl.ds(i * _EMB_DIM, _EMB_DIM)],
                wsem,
            )
        for i in range(_G):
            wd[i].wait()

    return sc_embed


def kernel(sparse_inputs, tables):
    idx_t = sparse_inputs.astype(jnp.int32).T
    sc = _build_sc_kernel()
    outs = []
    for g in range(_N_FIELDS // _G):
        tab_g = tables[g * _G:(g + 1) * _G].reshape(_G * _VOCAB, _EMB_DIM)
        idx_g = idx_t[g * _G:(g + 1) * _G]
        outs.append(sc(idx_g, tab_g))
    return jnp.concatenate(outs, axis=1)
